# baseline (speedup 1.0000x reference)
import jax
import jax.numpy as jnp
from jax.experimental import pallas as pl
from jax.experimental.pallas import tpu as pltpu

_BK = 400
_BK23 = 1000


def _dot(a, b, dims):
    return jax.lax.dot_general(a, b, (dims, ((), ())),
                               preferred_element_type=jnp.float32)


def _pass1_kernel(x0_ref, a_ref, wit_ref, bi_ref,
                  x_ref, abf_ref, x1bf_ref, x1t_ref):
    k = pl.program_id(0)
    x_blk = _dot(x0_ref[...], wit_ref[...], ((1,), (0,))) + bi_ref[0, :]
    x_ref[...] = x_blk
    abf_ref[...] = a_ref[...].astype(jnp.bfloat16)

    @pl.when(k == 0)
    def _():
        x1t_ref[...] = jnp.zeros_like(x1t_ref)

    x1t_ref[...] += _dot(x_blk.astype(jnp.bfloat16), abf_ref[...],
                         ((0,), (0,)))

    @pl.when(k == pl.num_programs(0) - 1)
    def _():
        x1bf_ref[...] = x1t_ref[...].T.astype(jnp.bfloat16)


def _pass23_kernel(abf_ref, x_ref, x1bf_ref, w1t_ref, b1_ref, w2t_ref,
                   b2_ref, xo_ref, x1l2_ref,
                   x12t_ref, x12bf_ref, xl1_ref):
    p = pl.program_id(0)
    i = pl.program_id(1)

    @pl.when(p == 0)
    def _():
        m = _dot(abf_ref[...], x1bf_ref[...], ((1,), (0,)))
        xn = _dot(x_ref[...] + m, w1t_ref[...], ((1,), (0,))) + b1_ref[0, :]
        xn = jnp.maximum(xn, 0.0)
        xl1_ref[pl.ds(i * _BK23, _BK23), :] = xn

        @pl.when(i == 0)
        def _():
            x12t_ref[...] = jnp.zeros_like(x12t_ref)

        x12t_ref[...] += _dot(xn.astype(jnp.bfloat16), abf_ref[...],
                              ((0,), (0,)))

        @pl.when(i == pl.num_programs(1) - 1)
        def _():
            x12 = x12t_ref[...].T
            x1l2_ref[...] = x12
            x12bf_ref[...] = x12.astype(jnp.bfloat16)

    @pl.when(p == 1)
    def _():
        m = _dot(abf_ref[...], x12bf_ref[...], ((1,), (0,)))
        xl1 = xl1_ref[pl.ds(i * _BK23, _BK23), :]
        xn = _dot(xl1 + m, w2t_ref[...], ((1,), (0,))) + b2_ref[0, :]
        xo_ref[...] = jnp.maximum(xn, 0.0)


def kernel(x_0, incidence_1, W_init, b_init, W1, b1, W2, b2):
    n, in_ch = x_0.shape
    hid = W_init.shape[0]
    n_edges = incidence_1.shape[1]
    steps = n // _BK
    steps23 = n // _BK23

    bi = b_init.reshape(1, hid)
    b1r = b1.reshape(1, hid)
    b2r = b2.reshape(1, hid)

    x_l0, a_bf, x1_bf = pl.pallas_call(
        _pass1_kernel,
        grid=(steps,),
        in_specs=[
            pl.BlockSpec((_BK, in_ch), lambda k: (k, 0)),
            pl.BlockSpec((_BK, n_edges), lambda k: (k, 0)),
            pl.BlockSpec((in_ch, hid), lambda k: (0, 0)),
            pl.BlockSpec((1, hid), lambda k: (0, 0)),
        ],
        out_specs=[
            pl.BlockSpec((_BK, hid), lambda k: (k, 0)),
            pl.BlockSpec((_BK, n_edges), lambda k: (k, 0)),
            pl.BlockSpec((n_edges, hid), lambda k: (0, 0)),
        ],
        out_shape=[
            jax.ShapeDtypeStruct((n, hid), jnp.float32),
            jax.ShapeDtypeStruct((n, n_edges), jnp.bfloat16),
            jax.ShapeDtypeStruct((n_edges, hid), jnp.bfloat16),
        ],
        scratch_shapes=[pltpu.VMEM((hid, n_edges), jnp.float32)],
        compiler_params=pltpu.CompilerParams(
            fuse_transposed_lhs_in_matmul=True),
    )(x_0, incidence_1, W_init.T, bi)

    x_out, x1_l2 = pl.pallas_call(
        _pass23_kernel,
        grid=(2, steps23),
        in_specs=[
            pl.BlockSpec((_BK23, n_edges), lambda p, i: (i, 0)),
            pl.BlockSpec((_BK23, hid), lambda p, i: (i, 0)),
            pl.BlockSpec((n_edges, hid), lambda p, i: (0, 0)),
            pl.BlockSpec((hid, hid), lambda p, i: (0, 0)),
            pl.BlockSpec((1, hid), lambda p, i: (0, 0)),
            pl.BlockSpec((hid, hid), lambda p, i: (0, 0)),
            pl.BlockSpec((1, hid), lambda p, i: (0, 0)),
        ],
        out_specs=[
            pl.BlockSpec((_BK23, hid), lambda p, i: (i * p, 0)),
            pl.BlockSpec((n_edges, hid), lambda p, i: (0, 0)),
        ],
        out_shape=[
            jax.ShapeDtypeStruct((n, hid), jnp.float32),
            jax.ShapeDtypeStruct((n_edges, hid), jnp.float32),
        ],
        scratch_shapes=[
            pltpu.VMEM((hid, n_edges), jnp.float32),
            pltpu.VMEM((n_edges, hid), jnp.bfloat16),
            pltpu.VMEM((n, hid), jnp.float32),
        ],
        compiler_params=pltpu.CompilerParams(
            fuse_transposed_lhs_in_matmul=True),
    )(a_bf, x_l0, x1_bf, W1.T, b1r, W2.T, b2r)

    return x_out, x1_l2


# speedup vs baseline: 1.0366x; 1.0366x over previous
import jax
import jax.numpy as jnp
from jax.experimental import pallas as pl
from jax.experimental.pallas import tpu as pltpu

_BK = 400
_BK23 = 1000


def _dot(a, b, dims):
    return jax.lax.dot_general(a, b, (dims, ((), ())),
                               preferred_element_type=jnp.float32)


def _pass1_kernel(x0_ref, a_ref, wit_ref, bi_ref,
                  x_ref, abf_ref, x1bf_ref, x1t_ref):
    k = pl.program_id(0)
    x_blk = _dot(x0_ref[...], wit_ref[...], ((1,), (0,))) + bi_ref[0, :]
    x_ref[...] = x_blk
    abf_ref[...] = a_ref[...].astype(jnp.bfloat16)

    @pl.when(k == 0)
    def _():
        x1t_ref[...] = jnp.zeros_like(x1t_ref)

    x1t_ref[...] += _dot(x_blk.astype(jnp.bfloat16), abf_ref[...],
                         ((0,), (0,)))

    @pl.when(k == pl.num_programs(0) - 1)
    def _():
        x1bf_ref[...] = x1t_ref[...].T.astype(jnp.bfloat16)


def _pass23_kernel(abf_ref, x_ref, x1bf_ref, w1t_ref, b1_ref, w2t_ref,
                   b2_ref, xo_ref, x1l2_ref,
                   x12t_ref, x12bf_ref, xl1_ref):
    p = pl.program_id(0)
    i = pl.program_id(1)

    @pl.when(p == 0)
    def _():
        m = _dot(abf_ref[...], x1bf_ref[...], ((1,), (0,)))
        xn = _dot(x_ref[...] + m, w1t_ref[...], ((1,), (0,))) + b1_ref[0, :]
        xn = jnp.maximum(xn, 0.0)
        xl1_ref[pl.ds(i * _BK23, _BK23), :] = xn

        @pl.when(i == 0)
        def _():
            x12t_ref[...] = jnp.zeros_like(x12t_ref)

        x12t_ref[...] += _dot(xn.astype(jnp.bfloat16), abf_ref[...],
                              ((0,), (0,)))

        @pl.when(i == pl.num_programs(1) - 1)
        def _():
            x12 = x12t_ref[...].T
            x1l2_ref[...] = x12
            x12bf_ref[...] = x12.astype(jnp.bfloat16)

    @pl.when(p == 1)
    def _():
        m = _dot(abf_ref[...], x12bf_ref[...], ((1,), (0,)))
        xl1 = xl1_ref[pl.ds(i * _BK23, _BK23), :]
        xn = _dot(xl1 + m, w2t_ref[...], ((1,), (0,))) + b2_ref[0, :]
        xo_ref[...] = jnp.maximum(xn, 0.0)


def kernel(x_0, incidence_1, W_init, b_init, W1, b1, W2, b2):
    n, in_ch = x_0.shape
    hid = W_init.shape[0]
    n_edges = incidence_1.shape[1]
    steps = n // _BK
    steps23 = n // _BK23

    bi = b_init.reshape(1, hid)
    b1r = b1.reshape(1, hid)
    b2r = b2.reshape(1, hid)

    x_l0, a_bf, x1_bf = pl.pallas_call(
        _pass1_kernel,
        grid=(steps,),
        in_specs=[
            pl.BlockSpec((_BK, in_ch), lambda k: (k, 0)),
            pl.BlockSpec((_BK, n_edges), lambda k: (k, 0)),
            pl.BlockSpec((in_ch, hid), lambda k: (0, 0)),
            pl.BlockSpec((1, hid), lambda k: (0, 0)),
        ],
        out_specs=[
            pl.BlockSpec((_BK, hid), lambda k: (k, 0)),
            pl.BlockSpec((_BK, n_edges), lambda k: (k, 0)),
            pl.BlockSpec((n_edges, hid), lambda k: (0, 0)),
        ],
        out_shape=[
            jax.ShapeDtypeStruct((n, hid), jnp.float32),
            jax.ShapeDtypeStruct((n, n_edges), jnp.bfloat16),
            jax.ShapeDtypeStruct((n_edges, hid), jnp.bfloat16),
        ],
        scratch_shapes=[pltpu.VMEM((hid, n_edges), jnp.float32)],
    )(x_0, incidence_1, W_init.T, bi)

    x_out, x1_l2 = pl.pallas_call(
        _pass23_kernel,
        grid=(2, steps23),
        in_specs=[
            pl.BlockSpec((_BK23, n_edges), lambda p, i: (i, 0)),
            pl.BlockSpec((_BK23, hid), lambda p, i: (i, 0)),
            pl.BlockSpec((n_edges, hid), lambda p, i: (0, 0)),
            pl.BlockSpec((hid, hid), lambda p, i: (0, 0)),
            pl.BlockSpec((1, hid), lambda p, i: (0, 0)),
            pl.BlockSpec((hid, hid), lambda p, i: (0, 0)),
            pl.BlockSpec((1, hid), lambda p, i: (0, 0)),
        ],
        out_specs=[
            pl.BlockSpec((_BK23, hid), lambda p, i: (i * p, 0)),
            pl.BlockSpec((n_edges, hid), lambda p, i: (0, 0)),
        ],
        out_shape=[
            jax.ShapeDtypeStruct((n, hid), jnp.float32),
            jax.ShapeDtypeStruct((n_edges, hid), jnp.float32),
        ],
        scratch_shapes=[
            pltpu.VMEM((hid, n_edges), jnp.float32),
            pltpu.VMEM((n_edges, hid), jnp.bfloat16),
            pltpu.VMEM((n, hid), jnp.float32),
        ],
    )(a_bf, x_l0, x1_bf, W1.T, b1r, W2.T, b2r)

    return x_out, x1_l2
